# baseline (speedup 1.0000x reference)
import functools

import jax
import jax.numpy as jnp
from jax import lax
from jax.experimental import pallas as pl
from jax.experimental.pallas import tpu as pltpu
from jax.experimental.pallas import tpu_sc as plsc

_B, _H, _W, _D = 8, 512, 512, 19
_NC, _NS, _L = 2, 16, 16
_NW = _NC * _NS
_WPB = _NW // _B
_RPW = _H // _WPB
_SR = 8
_NBAND = _RPW // _SR
_D0 = 10
_D1 = _D - _D0


def _sc_body(x_hbm, out_hbm, x_v0, x_v1, o_v0, o_v1,
             sem_i0, sem_i1, sem_o0, sem_o1):
    wid = lax.axis_index("s") * _NC + lax.axis_index("c")
    b = wid // _WPB
    row0 = (wid % _WPB) * _RPW
    xbufs = ((x_v0, sem_i0), (x_v1, sem_i1))
    obufs = ((o_v0, sem_o0, 0, _D0), (o_v1, sem_o1, _D0, _D1))

    def in_slice(t):
        return x_hbm.at[b, pl.ds(row0 + t * _SR, _SR), :]

    def out_slice(t, d0, nd):
        return out_hbm.at[b, pl.ds(d0, nd), pl.ds(row0 + t * _SR, _SR), :]

    def compute(x_v, o_v, d0, nd):
        def vec(i, carry):
            for srow in range(_SR):
                x = x_v[srow, pl.ds(i * _L, _L)]
                for d in range(nd):
                    o_v[d, srow, pl.ds(i * _L, _L)] = jnp.where(
                        x == d0 + d, jnp.float32(1.0), jnp.float32(0.0))
            return carry

        lax.fori_loop(0, _W // _L, vec, 0)

    def band(t, x_v, first):
        for o_v, sem_o, d0, nd in obufs:
            if not first:
                pltpu.make_async_copy(
                    o_v, out_slice(t - 1, d0, nd), sem_o).wait()
            compute(x_v, o_v, d0, nd)
            pltpu.async_copy(o_v, out_slice(t, d0, nd), sem_o)

    pltpu.sync_copy(in_slice(0), x_v0)
    pltpu.async_copy(in_slice(1), x_v1, sem_i1)
    band(0, x_v0, True)

    def pair(p, carry):
        for k in (1, 0):
            x_v, sem_i = xbufs[k]
            tt = 2 * p + 1 + (1 - k)
            pltpu.make_async_copy(in_slice(tt), x_v, sem_i).wait()
            pltpu.async_copy(in_slice(tt + 1), xbufs[1 - k][0],
                             xbufs[1 - k][1])
            band(tt, x_v, False)
        return carry

    lax.fori_loop(0, (_NBAND - 2) // 2, pair, 0)

    pltpu.make_async_copy(in_slice(_NBAND - 1), x_v1, sem_i1).wait()
    band(_NBAND - 1, x_v1, False)

    for o_v, sem_o, d0, nd in obufs:
        pltpu.make_async_copy(o_v, out_slice(_NBAND - 1, d0, nd), sem_o).wait()


@jax.jit
def _one_hot_sc(x):
    mesh = plsc.VectorSubcoreMesh(core_axis_name="c", subcore_axis_name="s")
    f = functools.partial(
        pl.kernel,
        out_type=jax.ShapeDtypeStruct((_B, _D, _H, _W), jnp.float32),
        mesh=mesh,
        compiler_params=pltpu.CompilerParams(use_tc_tiling_on_sc=True),
        scratch_types=[
            pltpu.VMEM((_SR, _W), jnp.int32),
            pltpu.VMEM((_SR, _W), jnp.int32),
            pltpu.VMEM((_D0, _SR, _W), jnp.float32),
            pltpu.VMEM((_D1, _SR, _W), jnp.float32),
            pltpu.SemaphoreType.DMA,
            pltpu.SemaphoreType.DMA,
            pltpu.SemaphoreType.DMA,
            pltpu.SemaphoreType.DMA,
        ],
    )(_sc_body)
    return f(x)


def kernel(X_in, ones):
    del ones
    return _one_hot_sc(X_in)
